# baseline (speedup 1.0000x reference)
import functools

import jax
import jax.numpy as jnp
from jax import lax
from jax.experimental import pallas as pl
from jax.experimental.pallas import tpu as pltpu
from jax.experimental.pallas import tpu_sc as plsc

_DIM = 64
_L = 16
_GC = 128
_BLK = 8192
_B8 = _BLK // 8


def _pack_table(weight_t):
    n = weight_t.shape[1]
    ng = (n + _BLK - 1) // _BLK

    def body(x_ref, o_ref):
        p = pltpu.bitcast(x_ref[...].astype(jnp.uint8), jnp.int32)
        pieces = [p[:, s * _B8:(s + 1) * _B8] for s in range(8)]
        o_ref[...] = jnp.swapaxes(jnp.concatenate(pieces, axis=0), 0, 1)

    return pl.pallas_call(
        body,
        grid=(ng,),
        in_specs=[pl.BlockSpec((_DIM, _BLK), lambda g: (0, g))],
        out_specs=pl.BlockSpec((_B8, 2 * _DIM), lambda g: (g, 0)),
        out_shape=jax.ShapeDtypeStruct((ng * _B8, 2 * _DIM), jnp.int32),
    )(weight_t)


def _make_kernel(batch, n_rows_packed, n_blocks_padded):
    info = plsc.get_sparse_core_info()
    nw = info.num_cores * info.num_subcores
    b_per_w = batch // nw
    nch = b_per_w // _GC
    mesh = plsc.VectorSubcoreMesh(core_axis_name="c", subcore_axis_name="s")

    @functools.partial(
        pl.kernel,
        mesh=mesh,
        out_type=jax.ShapeDtypeStruct((batch * _DIM,), jnp.float32),
        scratch_types=[
            pltpu.VMEM((nch, _GC), jnp.int32),
            pltpu.VMEM((nch, _GC), jnp.int32),
            pltpu.VMEM((b_per_w, 2 * _DIM), jnp.int32),
            pltpu.VMEM((256,), jnp.float32),
            pltpu.VMEM((n_blocks_padded,), jnp.float32),
            pltpu.VMEM((b_per_w,), jnp.float32),
            pltpu.VMEM((b_per_w * _DIM,), jnp.float32),
            pltpu.SemaphoreType.DMA,
        ],
        compiler_params=pltpu.CompilerParams(needs_layout_passes=False),
    )
    def k(idx_hbm, w_hbm, amax_hbm, code_hbm, out_hbm,
          idx_v, pidx_v, rows_v, code_v, amax_v, arow_v, out_f, sem):
        wid = lax.axis_index("s") * info.num_cores + lax.axis_index("c")
        base = wid * b_per_w

        for g in range(nch):
            pltpu.sync_copy(idx_hbm.at[pl.ds(base + g * _GC, _GC)],
                            idx_v.at[g])
        pltpu.sync_copy(code_hbm, code_v)

        m1023 = jnp.full((_L,), 1023, jnp.int32)
        for g in range(nch):
            for j in range(_GC // _L):
                iv = idx_v[g, pl.ds(j * _L, _L)]
                pv = jax.lax.shift_left(
                    jax.lax.shift_right_logical(iv, 13), 10) | (iv & m1023)
                pidx_v[g, pl.ds(j * _L, _L)] = pv

        copies = []
        for g in range(nch):
            copies.append(
                pltpu.async_copy(w_hbm.at[pidx_v.at[g]],
                                 rows_v.at[pl.ds(g * _GC, _GC)], sem))

        pltpu.sync_copy(amax_hbm, amax_v)

        for g in range(nch):
            for j in range(_GC // _L):
                iv = idx_v[g, pl.ds(j * _L, _L)]
                blk = jax.lax.shift_right_logical(iv, 6)
                arow_v[pl.ds(g * _GC + j * _L, _L)] = (
                    plsc.load_gather(amax_v, [blk]))

        iota4 = lax.iota(jnp.int32, _L) * 4
        mask = jnp.full((_L,), 255, jnp.int32)
        m7 = jnp.full((_L,), 7, jnp.int32)

        def grp_body(j, carry):
            row0 = j * _L
            av = arow_v[pl.ds(row0, _L)]
            g = j // (_GC // _L)
            jj = j % (_GC // _L)
            iv = idx_v[g, pl.ds(jj * _L, _L)]
            off = jax.lax.shift_left(
                jax.lax.shift_right_logical(iv, 10) & m7, 4)
            for r in range(_L):
                a = av[r]
                w = rows_v[row0 + r, pl.ds(off[r], _L)]
                ob = iota4 * b_per_w + (row0 + r)
                for b in range(4):
                    cw = jax.lax.shift_right_logical(w, 8 * b) & mask
                    v = plsc.load_gather(code_v, [cw]) * a
                    plsc.store_scatter(out_f, [ob + b * b_per_w], v)
            return carry

        jpg = _GC // _L
        for g in range(nch):
            copies[g].wait()
            lax.fori_loop(g * jpg, (g + 1) * jpg, grp_body, 0)

        ocopies = []
        for c in range(_DIM):
            ocopies.append(pltpu.async_copy(
                out_f.at[pl.ds(c * b_per_w, b_per_w)],
                out_hbm.at[pl.ds(c * batch + base, b_per_w)], sem))
        for c in ocopies:
            c.wait()

    return k


def kernel(input, weight, absmax, code):
    n_blocks = absmax.shape[0]
    pad = (-n_blocks) % _L
    amax_padded = jnp.concatenate(
        [absmax, jnp.zeros((pad,), absmax.dtype)]) if pad else absmax
    wp = _pack_table(weight.T)
    k = _make_kernel(input.shape[0], wp.shape[0], n_blocks + pad)
    out_flat = k(input, wp, amax_padded, code)
    return out_flat.reshape(_DIM, input.shape[0]).T


# speedup vs baseline: 1.0239x; 1.0239x over previous
import functools

import jax
import jax.numpy as jnp
from jax import lax
from jax.experimental import pallas as pl
from jax.experimental.pallas import tpu as pltpu
from jax.experimental.pallas import tpu_sc as plsc

_DIM = 64
_L = 16
_GC = 128
_BLK = 8192
_B8 = _BLK // 8


def _pack_table(weight_t):
    n = weight_t.shape[1]
    ng = (n + _BLK - 1) // _BLK

    def body(x_ref, o_ref):
        p = pltpu.bitcast(x_ref[...].astype(jnp.uint8), jnp.int32)
        pieces = [p[:, s * _B8:(s + 1) * _B8] for s in range(8)]
        o_ref[...] = jnp.swapaxes(jnp.concatenate(pieces, axis=0), 0, 1)

    return pl.pallas_call(
        body,
        grid=(ng,),
        in_specs=[pl.BlockSpec((_DIM, _BLK), lambda g: (0, g))],
        out_specs=pl.BlockSpec((_B8, 2 * _DIM), lambda g: (g, 0)),
        out_shape=jax.ShapeDtypeStruct((ng * _B8, 2 * _DIM), jnp.int32),
    )(weight_t)


def _make_kernel(batch, n_rows_packed, n_blocks_padded):
    info = plsc.get_sparse_core_info()
    nw = info.num_cores * info.num_subcores
    b_per_w = batch // nw
    nch = b_per_w // _GC
    mesh = plsc.VectorSubcoreMesh(core_axis_name="c", subcore_axis_name="s")

    @functools.partial(
        pl.kernel,
        mesh=mesh,
        out_type=jax.ShapeDtypeStruct((batch * _DIM,), jnp.float32),
        scratch_types=[
            pltpu.VMEM((nch, _GC), jnp.int32),
            pltpu.VMEM((nch, _GC), jnp.int32),
            pltpu.VMEM((b_per_w, 2 * _DIM), jnp.int32),
            pltpu.VMEM((256,), jnp.float32),
            pltpu.VMEM((n_blocks_padded,), jnp.float32),
            pltpu.VMEM((b_per_w,), jnp.float32),
            pltpu.VMEM((b_per_w * _DIM,), jnp.float32),
            pltpu.SemaphoreType.DMA,
        ],
        compiler_params=pltpu.CompilerParams(needs_layout_passes=False),
    )
    def k(idx_hbm, w_hbm, amax_hbm, code_hbm, out_hbm,
          idx_v, pidx_v, rows_v, code_v, amax_v, arow_v, out_f, sem):
        wid = lax.axis_index("s") * info.num_cores + lax.axis_index("c")
        base = wid * b_per_w

        for g in range(nch):
            pltpu.sync_copy(idx_hbm.at[pl.ds(base + g * _GC, _GC)],
                            idx_v.at[g])
        pltpu.sync_copy(code_hbm, code_v)

        m1023 = jnp.full((_L,), 1023, jnp.int32)
        for g in range(nch):
            for j in range(_GC // _L):
                iv = idx_v[g, pl.ds(j * _L, _L)]
                pv = jax.lax.shift_left(
                    jax.lax.shift_right_logical(iv, 13), 10) | (iv & m1023)
                pidx_v[g, pl.ds(j * _L, _L)] = pv

        copies = []
        for g in range(nch):
            copies.append(
                pltpu.async_copy(w_hbm.at[pidx_v.at[g]],
                                 rows_v.at[pl.ds(g * _GC, _GC)], sem))

        pltpu.sync_copy(amax_hbm, amax_v)

        for g in range(nch):
            for j in range(_GC // _L):
                iv = idx_v[g, pl.ds(j * _L, _L)]
                blk = jax.lax.shift_right_logical(iv, 6)
                arow_v[pl.ds(g * _GC + j * _L, _L)] = (
                    plsc.load_gather(amax_v, [blk]))

        iota4 = lax.iota(jnp.int32, _L) * 4
        mask = jnp.full((_L,), 255, jnp.int32)
        m7 = jnp.full((_L,), 7, jnp.int32)

        def grp_body(j, carry):
            row0 = j * _L
            av = arow_v[pl.ds(row0, _L)]
            g = j // (_GC // _L)
            jj = j % (_GC // _L)
            iv = idx_v[g, pl.ds(jj * _L, _L)]
            off = jax.lax.shift_left(
                jax.lax.shift_right_logical(iv, 10) & m7, 4)
            for r in range(_L):
                a = av[r]
                w = rows_v[row0 + r, pl.ds(off[r], _L)]
                ob = (row0 + r) * _DIM + iota4
                for b in range(4):
                    cw = jax.lax.shift_right_logical(w, 8 * b) & mask
                    v = plsc.load_gather(code_v, [cw]) * a
                    plsc.store_scatter(out_f, [ob + b], v)
            return carry

        jpg = _GC // _L
        for g in range(nch):
            copies[g].wait()
            lax.fori_loop(g * jpg, (g + 1) * jpg, grp_body, 0)

        pltpu.sync_copy(out_f, out_hbm.at[pl.ds(base * _DIM, b_per_w * _DIM)])

    return k


def kernel(input, weight, absmax, code):
    n_blocks = absmax.shape[0]
    pad = (-n_blocks) % _L
    amax_padded = jnp.concatenate(
        [absmax, jnp.zeros((pad,), absmax.dtype)]) if pad else absmax
    wp = _pack_table(weight.T)
    k = _make_kernel(input.shape[0], wp.shape[0], n_blocks + pad)
    out_flat = k(input, wp, amax_padded, code)
    return out_flat.reshape(input.shape[0], _DIM)


# speedup vs baseline: 1.0297x; 1.0057x over previous
import functools

import jax
import jax.numpy as jnp
from jax import lax
from jax.experimental import pallas as pl
from jax.experimental.pallas import tpu as pltpu
from jax.experimental.pallas import tpu_sc as plsc

_DIM = 64
_L = 16
_GC = 128
_BLK = 8192
_B8 = _BLK // 8


def _pack_table(weight_t):
    n = weight_t.shape[1]
    ng = (n + _BLK - 1) // _BLK

    def body(x_ref, o_ref):
        p = pltpu.bitcast(x_ref[...].astype(jnp.uint8), jnp.int32)
        pieces = [p[:, s * _B8:(s + 1) * _B8] for s in range(8)]
        o_ref[...] = jnp.swapaxes(jnp.concatenate(pieces, axis=0), 0, 1)

    return pl.pallas_call(
        body,
        grid=(ng,),
        in_specs=[pl.BlockSpec((_DIM, _BLK), lambda g: (0, g))],
        out_specs=pl.BlockSpec((_B8, 2 * _DIM), lambda g: (g, 0)),
        out_shape=jax.ShapeDtypeStruct((ng * _B8, 2 * _DIM), jnp.int32),
    )(weight_t)


def _make_kernel(batch, n_rows_packed, n_blocks_padded):
    info = plsc.get_sparse_core_info()
    nw = info.num_cores * info.num_subcores
    b_per_w = batch // nw
    nch = b_per_w // _GC
    mesh = plsc.VectorSubcoreMesh(core_axis_name="c", subcore_axis_name="s")

    @functools.partial(
        pl.kernel,
        mesh=mesh,
        out_type=jax.ShapeDtypeStruct((batch * _DIM,), jnp.float32),
        scratch_types=[
            pltpu.VMEM((nch, _GC), jnp.int32),
            pltpu.VMEM((nch, _GC), jnp.int32),
            pltpu.VMEM((b_per_w, 2 * _DIM), jnp.int32),
            pltpu.VMEM((256,), jnp.float32),
            pltpu.VMEM((n_blocks_padded,), jnp.float32),
            pltpu.VMEM((b_per_w,), jnp.float32),
            pltpu.VMEM((b_per_w * _DIM,), jnp.float32),
            pltpu.SemaphoreType.DMA,
        ],
        compiler_params=pltpu.CompilerParams(needs_layout_passes=False),
    )
    def k(idx_hbm, w_hbm, amax_hbm, code_hbm, out_hbm,
          idx_v, pidx_v, rows_v, code_v, amax_v, arow_v, out_f, sem):
        wid = lax.axis_index("s") * info.num_cores + lax.axis_index("c")
        base = wid * b_per_w

        for g in range(nch):
            pltpu.sync_copy(idx_hbm.at[pl.ds(base + g * _GC, _GC)],
                            idx_v.at[g])
        pltpu.sync_copy(code_hbm, code_v)

        m1023 = jnp.full((_L,), 1023, jnp.int32)
        for g in range(nch):
            for j in range(_GC // _L):
                iv = idx_v[g, pl.ds(j * _L, _L)]
                pv = jax.lax.shift_left(
                    jax.lax.shift_right_logical(iv, 13), 10) | (iv & m1023)
                pidx_v[g, pl.ds(j * _L, _L)] = pv

        copies = []
        for g in range(nch):
            copies.append(
                pltpu.async_copy(w_hbm.at[pidx_v.at[g]],
                                 rows_v.at[pl.ds(g * _GC, _GC)], sem))

        pltpu.sync_copy(amax_hbm, amax_v)

        for g in range(nch):
            for j in range(_GC // _L):
                iv = idx_v[g, pl.ds(j * _L, _L)]
                blk = jax.lax.shift_right_logical(iv, 6)
                arow_v[pl.ds(g * _GC + j * _L, _L)] = (
                    plsc.load_gather(amax_v, [blk]))

        for c in copies:
            c.wait()

        iota4 = lax.iota(jnp.int32, _L) * 4
        mask = jnp.full((_L,), 255, jnp.int32)
        m7 = jnp.full((_L,), 7, jnp.int32)

        def grp_body(j, carry):
            row0 = j * _L
            av = arow_v[pl.ds(row0, _L)]
            g = j // (_GC // _L)
            jj = j % (_GC // _L)
            iv = idx_v[g, pl.ds(jj * _L, _L)]
            off = jax.lax.shift_left(
                jax.lax.shift_right_logical(iv, 10) & m7, 4)
            for r in range(_L):
                a = av[r]
                w = rows_v[row0 + r, pl.ds(off[r], _L)]
                ob = (row0 + r) * _DIM + iota4
                for b in range(4):
                    cw = jax.lax.shift_right_logical(w, 8 * b) & mask
                    v = plsc.load_gather(code_v, [cw]) * a
                    plsc.store_scatter(out_f, [ob + b], v)
            return carry

        lax.fori_loop(0, b_per_w // _L, grp_body, 0)

        pltpu.sync_copy(out_f, out_hbm.at[pl.ds(base * _DIM, b_per_w * _DIM)])

    return k


def kernel(input, weight, absmax, code):
    n_blocks = absmax.shape[0]
    pad = (-n_blocks) % _L
    amax_padded = jnp.concatenate(
        [absmax, jnp.zeros((pad,), absmax.dtype)]) if pad else absmax
    wp = _pack_table(weight.T)
    k = _make_kernel(input.shape[0], wp.shape[0], n_blocks + pad)
    out_flat = k(input, wp, amax_padded, code)
    return out_flat.reshape(input.shape[0], _DIM)


# speedup vs baseline: 1.2333x; 1.1977x over previous
import functools

import jax
import jax.numpy as jnp
from jax import lax
from jax.experimental import pallas as pl
from jax.experimental.pallas import tpu as pltpu
from jax.experimental.pallas import tpu_sc as plsc

_DIM = 64
_L = 16
_GC = 128
_BLK = 16384
_B8 = _BLK // 8


def _pack_table(weight_t):
    n = weight_t.shape[1]
    ng = (n + _BLK - 1) // _BLK

    def body(x_ref, o_ref):
        p = pltpu.bitcast(x_ref[...].astype(jnp.uint8), jnp.int32)
        pieces = [p[:, s * _B8:(s + 1) * _B8] for s in range(8)]
        o_ref[...] = jnp.swapaxes(jnp.concatenate(pieces, axis=0), 0, 1)

    return pl.pallas_call(
        body,
        grid=(ng,),
        in_specs=[pl.BlockSpec((_DIM, _BLK), lambda g: (0, g))],
        out_specs=pl.BlockSpec((_B8, 2 * _DIM), lambda g: (g, 0)),
        out_shape=jax.ShapeDtypeStruct((ng * _B8, 2 * _DIM), jnp.int32),
    )(weight_t)


def _make_kernel(batch, n_rows_packed, n_blocks_padded):
    info = plsc.get_sparse_core_info()
    nw = info.num_cores * info.num_subcores
    b_per_w = batch // nw
    nch = b_per_w // _GC
    mesh = plsc.VectorSubcoreMesh(core_axis_name="c", subcore_axis_name="s")

    @functools.partial(
        pl.kernel,
        mesh=mesh,
        out_type=jax.ShapeDtypeStruct((batch * _DIM,), jnp.float32),
        scratch_types=[
            pltpu.VMEM((nch, _GC), jnp.int32),
            pltpu.VMEM((nch, _GC), jnp.int32),
            pltpu.VMEM((b_per_w, 2 * _DIM), jnp.int32),
            pltpu.VMEM((256,), jnp.float32),
            pltpu.VMEM((n_blocks_padded,), jnp.float32),
            pltpu.VMEM((b_per_w,), jnp.float32),
            pltpu.VMEM((b_per_w * _DIM,), jnp.float32),
            pltpu.SemaphoreType.DMA,
        ],
        compiler_params=pltpu.CompilerParams(needs_layout_passes=False),
    )
    def k(idx_hbm, w_hbm, amax_hbm, code_hbm, out_hbm,
          idx_v, pidx_v, rows_v, code_v, amax_v, arow_v, out_f, sem):
        wid = lax.axis_index("s") * info.num_cores + lax.axis_index("c")
        base = wid * b_per_w

        for g in range(nch):
            pltpu.sync_copy(idx_hbm.at[pl.ds(base + g * _GC, _GC)],
                            idx_v.at[g])
        pltpu.sync_copy(code_hbm, code_v)

        m1023 = jnp.full((_L,), _B8 - 1, jnp.int32)
        for g in range(nch):
            for j in range(_GC // _L):
                iv = idx_v[g, pl.ds(j * _L, _L)]
                pv = jax.lax.shift_left(
                    jax.lax.shift_right_logical(iv, 14), 11) | (iv & m1023)
                pidx_v[g, pl.ds(j * _L, _L)] = pv

        copies = []
        for g in range(nch):
            copies.append(
                pltpu.async_copy(w_hbm.at[pidx_v.at[g]],
                                 rows_v.at[pl.ds(g * _GC, _GC)], sem))

        pltpu.sync_copy(amax_hbm, amax_v)

        for g in range(nch):
            for j in range(_GC // _L):
                iv = idx_v[g, pl.ds(j * _L, _L)]
                blk = jax.lax.shift_right_logical(iv, 6)
                arow_v[pl.ds(g * _GC + j * _L, _L)] = (
                    plsc.load_gather(amax_v, [blk]))

        for c in copies:
            c.wait()

        iota4 = lax.iota(jnp.int32, _L) * 4
        mask = jnp.full((_L,), 255, jnp.int32)
        m7 = jnp.full((_L,), 7, jnp.int32)

        def grp_body(j, carry):
            row0 = j * _L
            av = arow_v[pl.ds(row0, _L)]
            g = j // (_GC // _L)
            jj = j % (_GC // _L)
            iv = idx_v[g, pl.ds(jj * _L, _L)]
            off = jax.lax.shift_left(
                jax.lax.shift_right_logical(iv, 11) & m7, 4)
            for r in range(_L):
                a = av[r]
                w = rows_v[row0 + r, pl.ds(off[r], _L)]
                ob = (row0 + r) * _DIM + iota4
                for b in range(4):
                    cw = jax.lax.shift_right_logical(w, 8 * b) & mask
                    v = plsc.load_gather(code_v, [cw]) * a
                    plsc.store_scatter(out_f, [ob + b], v)
            return carry

        lax.fori_loop(0, b_per_w // _L, grp_body, 0)

        pltpu.sync_copy(out_f, out_hbm.at[pl.ds(base * _DIM, b_per_w * _DIM)])

    return k


def kernel(input, weight, absmax, code):
    n_blocks = absmax.shape[0]
    pad = (-n_blocks) % _L
    amax_padded = jnp.concatenate(
        [absmax, jnp.zeros((pad,), absmax.dtype)]) if pad else absmax
    wp = _pack_table(weight.T)
    k = _make_kernel(input.shape[0], wp.shape[0], n_blocks + pad)
    out_flat = k(input, wp, amax_padded, code)
    return out_flat.reshape(input.shape[0], _DIM)
